# baseline (speedup 1.0000x reference)
import jax
import jax.numpy as jnp
import numpy as np
from jax.experimental import pallas as pl


def _perm(Z: int) -> np.ndarray:
    N = 16 * Z
    P = np.zeros((N, N), np.float32)
    for p in range(N):
        ch, m, z = p // (4 * Z), (p % (4 * Z)) // Z, p % Z
        j, k = ch // 2, ch % 2
        P[p, m * 4 * Z + j * 2 * Z + 2 * z + k] = 1.0
    return P


def _body(x_ref, p_ref, o_ref):
    H = o_ref.shape[1]
    W4 = o_ref.shape[3]
    for i in range(2):
        v = x_ref[0, i]
        c = jnp.concatenate([v[0], v[1], v[2], v[3]], axis=-1)
        y = jax.lax.dot(c, p_ref[...])
        o_ref[0, :, i, :, :] = y.reshape(H, W4, o_ref.shape[4])


def kernel(x):
    B, Cr3, H, W, Z = x.shape
    C = Cr3 // 8
    G = B * C
    HW = H * W
    xr = x.reshape(G, 2, 4, HW // 4, 4 * Z)
    P = jnp.asarray(_perm(Z))

    out = pl.pallas_call(
        _body,
        grid=(G,),
        in_specs=[
            pl.BlockSpec((1, 2, 4, HW // 4, 4 * Z), lambda g: (g, 0, 0, 0, 0)),
            pl.BlockSpec((16 * Z, 16 * Z), lambda g: (0, 0)),
        ],
        out_specs=pl.BlockSpec((1, H, 2, W // 4, 16 * Z), lambda g: (g, 0, 0, 0, 0)),
        out_shape=jax.ShapeDtypeStruct((G, H, 2, W // 4, 16 * Z), jnp.float32),
    )(xr, P)
    return out.reshape(B, C, 2 * H, 2 * W, 2 * Z)


# speedup vs baseline: 1.0855x; 1.0855x over previous
import jax
import jax.numpy as jnp
import numpy as np
from jax.experimental import pallas as pl
from jax.experimental.pallas import tpu as pltpu

_K = 8


def _perm(Z: int) -> np.ndarray:
    N = 16 * Z
    P = np.zeros((N, N), np.float32)
    for p in range(N):
        ch, m, z = p // (4 * Z), (p % (4 * Z)) // Z, p % Z
        j, k = ch // 2, ch % 2
        P[p, m * 4 * Z + j * 2 * Z + 2 * z + k] = 1.0
    return P


def _make_body(G, H, W4, N):
    def body(x_hbm, p_ref, o_hbm, in_bufs, out_bufs, in_sems, out_sems):
        def in_copy(g, s):
            return pltpu.make_async_copy(x_hbm.at[g], in_bufs.at[s], in_sems.at[s])

        def out_copy(g, s):
            return pltpu.make_async_copy(out_bufs.at[s], o_hbm.at[g], out_sems.at[s])

        for s in range(_K):
            in_copy(s, s).start()

        niter = G // _K

        def outer(it, carry):
            base = it * _K
            for s in range(_K):
                g = base + s
                in_copy(g, s).wait()

                @pl.when(it > 0)
                def _wait_prev():
                    out_copy(g - _K, s).wait()

                v = in_bufs[s]
                for i in range(2):
                    c = jnp.concatenate(
                        [v[i, 0], v[i, 1], v[i, 2], v[i, 3]], axis=-1
                    )
                    y = jax.lax.dot(c, p_ref[...])
                    out_bufs[s, :, i, :, :] = y.reshape(H, W4, N)
                out_copy(g, s).start()

                @pl.when(it < niter - 1)
                def _next_in():
                    in_copy(g + _K, s).start()
            return carry

        jax.lax.fori_loop(0, niter, outer, 0)
        for s in range(_K):
            out_copy(G - _K + s, s).wait()

    return body


def kernel(x):
    B, Cr3, H, W, Z = x.shape
    C = Cr3 // 8
    G = B * C
    HW = H * W
    N = 16 * Z
    xr = x.reshape(G, 2, 4, HW // 4, 4 * Z)
    P = jnp.asarray(_perm(Z))

    out = pl.pallas_call(
        _make_body(G, H, W // 4, N),
        in_specs=[
            pl.BlockSpec(memory_space=pl.ANY),
            pl.BlockSpec(memory_space=pltpu.VMEM),
        ],
        out_specs=pl.BlockSpec(memory_space=pl.ANY),
        out_shape=jax.ShapeDtypeStruct((G, H, 2, W // 4, N), jnp.float32),
        scratch_shapes=[
            pltpu.VMEM((_K, 2, 4, HW // 4, 4 * Z), jnp.float32),
            pltpu.VMEM((_K, H, 2, W // 4, N), jnp.float32),
            pltpu.SemaphoreType.DMA((_K,)),
            pltpu.SemaphoreType.DMA((_K,)),
        ],
    )(xr, P)
    return out.reshape(B, C, 2 * H, 2 * W, 2 * Z)
